# Initial kernel scaffold; baseline (speedup 1.0000x reference)
import jax
import jax.numpy as jnp
from jax.experimental import pallas as pl


def kernel(graph, tensor, batch_list, weight, bias, mean_scale):
    raise NotImplementedError("write your pallas kernel here")


# speedup vs baseline: 3.7336x; 3.7336x over previous
import functools

import jax
import jax.numpy as jnp
from jax import lax
from jax.experimental import pallas as pl
from jax.experimental.pallas import tpu as pltpu
from jax.experimental.pallas import tpu_sc as plsc

NC = 2
NS = 16
L = 16
GPAD = 112


def _sload(ref, i):
    return ref[pl.ds(i, L)][0]


def _searchsorted_right(ends_v, row, b):
    def step(k, lo):
        sz = b >> (k + 1)
        probe = lo + sz - 1
        go = _sload(ends_v, probe) <= row
        return lo + jnp.where(go, sz, 0)

    return lax.fori_loop(0, b.bit_length() - 1, step, jnp.int32(0))


def _advance(g, e, row, ends_v, nxt_v):
    adv = row >= e
    gn = _sload(nxt_v, g)
    en = _sload(ends_v, gn)
    g = jnp.where(adv, gn, g)
    e = jnp.where(adv, en, e)
    return g, e


def _make_pass1(n, d, b, rpw, chunk):
    d2 = 2 * d
    nfull = rpw // chunk
    rem = rpw % chunk
    bsh = b // NS
    mesh = plsc.VectorSubcoreMesh(core_axis_name="c", subcore_axis_name="s")

    @functools.partial(
        pl.kernel,
        out_type=jax.ShapeDtypeStruct((NC * NS, GPAD, d2), jnp.float32),
        mesh=mesh,
        scratch_types=[
            pltpu.VMEM((chunk, d), jnp.float32),
            pltpu.VMEM((GPAD, d2), jnp.float32),
            pltpu.VMEM((b + L,), jnp.int32),
            pltpu.VMEM((b + L,), jnp.int32),
        ],
    )
    def pass1(tensor_hbm, ends_hbm, nxt_hbm, part_hbm, buf, acc, ends_v, nxt_v):
        cid = lax.axis_index("c")
        sid = lax.axis_index("s")
        wid = cid * NS + sid
        zero16 = jnp.zeros((L,), jnp.float32)

        pltpu.sync_copy(ends_hbm, ends_v.at[pl.ds(0, b)])
        pltpu.sync_copy(nxt_hbm, nxt_v.at[pl.ds(0, b)])

        def arow(r, carry):
            for j in range(d2 // L):
                acc[r, pl.ds(L * j, L)] = zero16
            return carry

        lax.fori_loop(0, GPAD, arow, 0)

        row0 = wid * rpw
        g0 = _searchsorted_right(ends_v, row0, b)
        g0 = pl.multiple_of(jnp.minimum(g0 & ~7, b - GPAD), 8)

        def rows_body(base, nrows, carry):
            def do_row(r, carry2):
                g, e = _advance(carry2[0], carry2[1], base + r, ends_v, nxt_v)
                gl = jnp.minimum(g - g0, GPAD - 1)
                for j in range(d // L):
                    v = buf[r, pl.ds(L * j, L)]
                    plsc.addupdate(acc.at[gl, pl.ds(L * j, L)], v)
                    plsc.addupdate(acc.at[gl, pl.ds(d + L * j, L)], v * v)
                return (g, e)

            return lax.fori_loop(0, nrows, do_row, carry)

        def do_chunk(i, carry):
            base = row0 + i * chunk
            pltpu.sync_copy(tensor_hbm.at[pl.ds(base, chunk)], buf)
            return rows_body(base, chunk, carry)

        gstart = _searchsorted_right(ends_v, row0, b)
        e0 = _sload(ends_v, gstart)
        carry = lax.fori_loop(0, nfull, do_chunk, (gstart, e0))
        if rem:
            base = row0 + nfull * chunk
            pltpu.sync_copy(tensor_hbm.at[pl.ds(base, rem)],
                            buf.at[pl.ds(0, rem)])
            rows_body(base, rem, carry)

        pltpu.sync_copy(acc, part_hbm.at[wid])

    return pass1


def _make_pass2(n, d, b, rpw, chunk):
    nfull = rpw // chunk
    rem = rpw % chunk
    mesh = plsc.VectorSubcoreMesh(core_axis_name="c", subcore_axis_name="s")

    @functools.partial(
        pl.kernel,
        out_type=jax.ShapeDtypeStruct((n, d), jnp.float32),
        mesh=mesh,
        scratch_types=[
            pltpu.VMEM((chunk, d), jnp.float32),
            pltpu.VMEM((chunk, d), jnp.float32),
            pltpu.VMEM((GPAD, d), jnp.float32),
            pltpu.VMEM((GPAD, d), jnp.float32),
            pltpu.VMEM((b + L,), jnp.int32),
            pltpu.VMEM((b + L,), jnp.int32),
        ],
    )
    def pass2(tensor_hbm, ends_hbm, nxt_hbm, alpha_hbm, beta_hbm, out_hbm,
              buf, obuf, al, be, ends_v, nxt_v):
        cid = lax.axis_index("c")
        sid = lax.axis_index("s")
        wid = cid * NS + sid

        pltpu.sync_copy(ends_hbm, ends_v.at[pl.ds(0, b)])
        pltpu.sync_copy(nxt_hbm, nxt_v.at[pl.ds(0, b)])
        row0 = wid * rpw
        g0 = _searchsorted_right(ends_v, row0, b)
        g0c = pl.multiple_of(jnp.minimum(g0 & ~7, b - GPAD), 8)
        pltpu.sync_copy(alpha_hbm.at[pl.ds(g0c, GPAD)], al)
        pltpu.sync_copy(beta_hbm.at[pl.ds(g0c, GPAD)], be)

        def rows_body(base, nrows, carry):
            def do_row(r, carry2):
                g, e = _advance(carry2[0], carry2[1], base + r, ends_v, nxt_v)
                gl = g - g0c
                for j in range(d // L):
                    v = buf[r, pl.ds(L * j, L)]
                    a = al[gl, pl.ds(L * j, L)]
                    bb = be[gl, pl.ds(L * j, L)]
                    obuf[r, pl.ds(L * j, L)] = v * a + bb
                return (g, e)

            return lax.fori_loop(0, nrows, do_row, carry)

        def do_chunk(i, carry):
            base = row0 + i * chunk
            pltpu.sync_copy(tensor_hbm.at[pl.ds(base, chunk)], buf)
            carry = rows_body(base, chunk, carry)
            pltpu.sync_copy(obuf, out_hbm.at[pl.ds(base, chunk)])
            return carry

        e0 = _sload(ends_v, g0)
        carry = lax.fori_loop(0, nfull, do_chunk, (g0, e0))
        if rem:
            base = row0 + nfull * chunk
            pltpu.sync_copy(tensor_hbm.at[pl.ds(base, rem)],
                            buf.at[pl.ds(0, rem)])
            rows_body(base, rem, carry)
            pltpu.sync_copy(obuf.at[pl.ds(0, rem)],
                            out_hbm.at[pl.ds(base, rem)])

    return pass2


def _finalize_tc(partials, gtarg, counts, weight, bias, mean_scale, b):
    wg, d2 = partials.shape[0] * partials.shape[1], partials.shape[2]
    d = d2 // 2

    def body(part_ref, gt_ref, cnt_ref, w_ref, b_ref, ms_ref, alpha_ref, beta_ref):
        gidx = lax.broadcasted_iota(jnp.int32, (wg, b), 1)
        onehot = (gt_ref[...] == gidx).astype(jnp.float32)
        part2d = part_ref[...].reshape(wg, d2)
        s = lax.dot_general(onehot, part2d, (((0,), (0,)), ((), ())),
                            preferred_element_type=jnp.float32)
        sx = s[:, :d]
        sx2 = s[:, d:]
        invn = 1.0 / cnt_ref[...]
        mean = sx * invn
        c = mean * ms_ref[...]
        var = sx2 * invn - 2.0 * c * mean + c * c
        alpha = w_ref[...] * lax.rsqrt(var + 1e-6)
        alpha_ref[...] = alpha
        beta_ref[...] = b_ref[...] - alpha * c

    return pl.pallas_call(
        body,
        out_shape=[
            jax.ShapeDtypeStruct((b, d), jnp.float32),
            jax.ShapeDtypeStruct((b, d), jnp.float32),
        ],
    )(partials, gtarg, counts, weight, bias, mean_scale)


def kernel(graph, tensor, batch_list, weight, bias, mean_scale):
    n, d = tensor.shape
    b = batch_list.shape[0]
    w = NC * NS
    rpw = n // w
    assert n == w * rpw and rpw % 8 == 0
    chunk = 256

    ends = jnp.cumsum(batch_list.astype(jnp.int32))
    ne_idx = jnp.where(batch_list > 0, jnp.arange(b, dtype=jnp.int32), b - 1)
    sufmin = lax.cummin(ne_idx[::-1])[::-1]
    nxt = jnp.concatenate([sufmin[1:], jnp.full((1,), b - 1, jnp.int32)])
    partials = _make_pass1(n, d, b, rpw, chunk)(tensor, ends, nxt)
    g0 = jnp.searchsorted(ends, jnp.arange(w, dtype=jnp.int32) * rpw,
                          side='right').astype(jnp.int32)
    g0 = jnp.minimum(g0 & ~7, b - GPAD)
    gtarg = (g0[:, None] + jnp.arange(GPAD, dtype=jnp.int32)[None, :]).reshape(-1, 1)
    counts = batch_list.astype(jnp.float32).reshape(b, 1)
    alpha, beta = _finalize_tc(
        partials, gtarg, counts,
        weight.reshape(1, d), bias.reshape(1, d), mean_scale.reshape(1, d), b)
    return _make_pass2(n, d, b, rpw, chunk)(tensor, ends, nxt, alpha, beta)


# speedup vs baseline: 7.9798x; 2.1373x over previous
import functools

import jax
import jax.numpy as jnp
from jax import lax
from jax.experimental import pallas as pl
from jax.experimental.pallas import tpu as pltpu
from jax.experimental.pallas import tpu_sc as plsc

NC = 2
NS = 16
L = 16
GPAD = 112
NJ = 8
BLK = 16


def _sload(ref, i):
    return ref[pl.ds(i, L)][0]


def _searchsorted_right(ends_v, row, b):
    def step(k, lo):
        sz = b >> (k + 1)
        probe = lo + sz - 1
        go = _sload(ends_v, probe) <= row
        return lo + jnp.where(go, sz, 0)

    return lax.fori_loop(0, b.bit_length() - 1, step, jnp.int32(0))


def _hop(g, e, row, ends_v, nxt_v):
    adv = row >= e
    gn = _sload(nxt_v, g)
    en = _sload(ends_v, gn)
    return jnp.where(adv, gn, g), jnp.where(adv, en, e)


def _make_pass1(n, d, b, rpw, chunk):
    d2 = 2 * d
    nfull = rpw // chunk
    rem = rpw % chunk
    nblk = chunk // BLK
    assert nfull % 2 == 0 and chunk % BLK == 0 and rem % 8 == 0
    mesh = plsc.VectorSubcoreMesh(core_axis_name="c", subcore_axis_name="s")

    @functools.partial(
        pl.kernel,
        out_type=jax.ShapeDtypeStruct((NC * NS, GPAD, d2), jnp.float32),
        mesh=mesh,
        scratch_types=[
            pltpu.VMEM((chunk, d), jnp.float32),
            pltpu.VMEM((chunk, d), jnp.float32),
            pltpu.VMEM((GPAD, d2), jnp.float32),
            pltpu.VMEM((b + L,), jnp.int32),
            pltpu.VMEM((b + L,), jnp.int32),
            pltpu.VMEM((2 * L,), jnp.int32),
            pltpu.SemaphoreType.DMA,
            pltpu.SemaphoreType.DMA,
        ],
    )
    def pass1(tensor_hbm, ends_hbm, nxt_hbm, part_hbm,
              buf0, buf1, acc, ends_v, nxt_v, ptr, sem0, sem1):
        cid = lax.axis_index("c")
        sid = lax.axis_index("s")
        wid = cid * NS + sid
        zero16 = jnp.zeros((L,), jnp.float32)

        pltpu.sync_copy(ends_hbm, ends_v.at[pl.ds(0, b)])
        pltpu.sync_copy(nxt_hbm, nxt_v.at[pl.ds(0, b)])
        ptr[pl.ds(0, L)] = jnp.zeros((L,), jnp.int32)
        ptr[pl.ds(L, L)] = jnp.zeros((L,), jnp.int32)

        def arow(r, carry):
            for j in range(d2 // L):
                acc[r, pl.ds(L * j, L)] = zero16
            return carry

        lax.fori_loop(0, GPAD, arow, 0)

        row0 = wid * rpw
        gstart = _searchsorted_right(ends_v, row0, b)
        g0 = pl.multiple_of(jnp.minimum(gstart & ~7, b - GPAD), 8)

        def row_update(buf, rloc, row, gR, eR):
            gR, eR = _hop(gR, eR, row, ends_v, nxt_v)
            gl = jnp.minimum(gR - g0, GPAD - 1)
            for j in range(NJ):
                v = buf[rloc, pl.ds(L * j, L)]
                plsc.addupdate(acc.at[gl, pl.ds(L * j, L)], v)
                plsc.addupdate(acc.at[gl, pl.ds(d + L * j, L)], v * v)
            return gR, eR

        def blocks_body(buf, base, carry):
            def do_block(k, c2):
                gP, eP = c2
                rloc = k * BLK
                rb = base + rloc
                boundary = rb + BLK > eP
                glP = jnp.minimum(gP - g0, GPAD - 1)

                @pl.when(jnp.logical_not(boundary))
                def _fast():
                    for j in range(NJ):
                        bs = buf[rloc, pl.ds(L * j, L)]
                        bq = bs * bs
                        for r in range(1, BLK):
                            v = buf[rloc + r, pl.ds(L * j, L)]
                            bs = bs + v
                            bq = bq + v * v
                        plsc.addupdate(acc.at[glP, pl.ds(L * j, L)], bs)
                        plsc.addupdate(acc.at[glP, pl.ds(d + L * j, L)], bq)

                @pl.when(boundary)
                def _slow():
                    gR, eR = gP, eP
                    for r in range(BLK):
                        gR, eR = row_update(buf, rloc + r, rb + r, gR, eR)
                    ptr[pl.ds(0, L)] = jnp.full((L,), gR, jnp.int32)
                    ptr[pl.ds(L, L)] = jnp.full((L,), eR, jnp.int32)

                gS = _sload(ptr, 0)
                eS = _sload(ptr, L)
                gP = jnp.where(boundary, gS, gP)
                eP = jnp.where(boundary, eS, eP)
                return _hop(gP, eP, rb + BLK, ends_v, nxt_v)

            return lax.fori_loop(0, nblk, do_block, carry)

        e0 = _sload(ends_v, gstart)
        carry = (gstart, e0)

        bufs = (buf0, buf1)
        sems = (sem0, sem1)
        for i in range(2):
            pltpu.async_copy(tensor_hbm.at[pl.ds(row0 + i * chunk, chunk)],
                             bufs[i], sems[i])

        def do_pair(p, carry):
            base = row0 + 2 * p * chunk
            for h in range(2):
                pltpu.make_async_copy(tensor_hbm.at[pl.ds(0, chunk)],
                                      bufs[h], sems[h]).wait()

                @pl.when(2 * p + 2 + h < nfull)
                def _prefetch():
                    pltpu.async_copy(
                        tensor_hbm.at[pl.ds(base + (2 + h) * chunk, chunk)],
                        bufs[h], sems[h])

                carry = blocks_body(bufs[h], base + h * chunk, carry)
            return carry

        carry = lax.fori_loop(0, nfull // 2, do_pair, carry)
        if rem:
            base = row0 + nfull * chunk
            pltpu.sync_copy(tensor_hbm.at[pl.ds(base, rem)],
                            buf0.at[pl.ds(0, rem)])

            def tail_row(r, c2):
                return row_update(buf0, r, base + r, c2[0], c2[1])

            lax.fori_loop(0, rem, tail_row, carry)

        pltpu.sync_copy(acc, part_hbm.at[wid])

    return pass1


def _make_pass2(n, d, b, rpw, chunk):
    nfull = rpw // chunk
    rem = rpw % chunk
    nblk = chunk // BLK
    assert nfull % 2 == 0 and chunk % BLK == 0 and rem % 8 == 0
    mesh = plsc.VectorSubcoreMesh(core_axis_name="c", subcore_axis_name="s")

    @functools.partial(
        pl.kernel,
        out_type=jax.ShapeDtypeStruct((n, d), jnp.float32),
        mesh=mesh,
        scratch_types=[
            pltpu.VMEM((chunk, d), jnp.float32),
            pltpu.VMEM((chunk, d), jnp.float32),
            pltpu.VMEM((chunk, d), jnp.float32),
            pltpu.VMEM((chunk, d), jnp.float32),
            pltpu.VMEM((GPAD, d), jnp.float32),
            pltpu.VMEM((GPAD, d), jnp.float32),
            pltpu.VMEM((b + L,), jnp.int32),
            pltpu.VMEM((b + L,), jnp.int32),
            pltpu.VMEM((2 * L,), jnp.int32),
            pltpu.SemaphoreType.DMA,
            pltpu.SemaphoreType.DMA,
            pltpu.SemaphoreType.DMA,
            pltpu.SemaphoreType.DMA,
        ],
    )
    def pass2(tensor_hbm, ends_hbm, nxt_hbm, alpha_hbm, beta_hbm, out_hbm,
              buf0, buf1, obuf0, obuf1, al, be, ends_v, nxt_v, ptr,
              sem0, sem1, osem0, osem1):
        cid = lax.axis_index("c")
        sid = lax.axis_index("s")
        wid = cid * NS + sid

        pltpu.sync_copy(ends_hbm, ends_v.at[pl.ds(0, b)])
        pltpu.sync_copy(nxt_hbm, nxt_v.at[pl.ds(0, b)])
        ptr[pl.ds(0, L)] = jnp.zeros((L,), jnp.int32)
        ptr[pl.ds(L, L)] = jnp.zeros((L,), jnp.int32)
        row0 = wid * rpw
        gstart = _searchsorted_right(ends_v, row0, b)
        g0c = pl.multiple_of(jnp.minimum(gstart & ~7, b - GPAD), 8)
        pltpu.sync_copy(alpha_hbm.at[pl.ds(g0c, GPAD)], al)
        pltpu.sync_copy(beta_hbm.at[pl.ds(g0c, GPAD)], be)

        def row_apply(buf, obuf, rloc, row, gR, eR):
            gR, eR = _hop(gR, eR, row, ends_v, nxt_v)
            gl = jnp.minimum(gR - g0c, GPAD - 1)
            for j in range(NJ):
                v = buf[rloc, pl.ds(L * j, L)]
                a = al[gl, pl.ds(L * j, L)]
                bb = be[gl, pl.ds(L * j, L)]
                obuf[rloc, pl.ds(L * j, L)] = v * a + bb
            return gR, eR

        def blocks_body(buf, obuf, base, carry):
            def do_block(k, c2):
                gP, eP = c2
                rloc = k * BLK
                rb = base + rloc
                boundary = rb + BLK > eP
                glP = jnp.minimum(gP - g0c, GPAD - 1)

                @pl.when(jnp.logical_not(boundary))
                def _fast():
                    a = [al[glP, pl.ds(L * j, L)] for j in range(NJ)]
                    bb = [be[glP, pl.ds(L * j, L)] for j in range(NJ)]
                    for r in range(BLK):
                        for j in range(NJ):
                            v = buf[rloc + r, pl.ds(L * j, L)]
                            obuf[rloc + r, pl.ds(L * j, L)] = v * a[j] + bb[j]

                @pl.when(boundary)
                def _slow():
                    gR, eR = gP, eP
                    for r in range(BLK):
                        gR, eR = row_apply(buf, obuf, rloc + r, rb + r, gR, eR)
                    ptr[pl.ds(0, L)] = jnp.full((L,), gR, jnp.int32)
                    ptr[pl.ds(L, L)] = jnp.full((L,), eR, jnp.int32)

                gS = _sload(ptr, 0)
                eS = _sload(ptr, L)
                gP = jnp.where(boundary, gS, gP)
                eP = jnp.where(boundary, eS, eP)
                return _hop(gP, eP, rb + BLK, ends_v, nxt_v)

            return lax.fori_loop(0, nblk, do_block, carry)

        e0 = _sload(ends_v, gstart)
        carry = (gstart, e0)

        bufs = (buf0, buf1)
        sems = (sem0, sem1)
        obufs = (obuf0, obuf1)
        osems = (osem0, osem1)
        for i in range(2):
            pltpu.async_copy(tensor_hbm.at[pl.ds(row0 + i * chunk, chunk)],
                             bufs[i], sems[i])

        def do_pair(p, carry):
            base = row0 + 2 * p * chunk
            for h in range(2):
                cbase = base + h * chunk
                pltpu.make_async_copy(tensor_hbm.at[pl.ds(0, chunk)],
                                      bufs[h], sems[h]).wait()

                @pl.when(2 * p + 2 + h < nfull)
                def _prefetch():
                    pltpu.async_copy(
                        tensor_hbm.at[pl.ds(base + (2 + h) * chunk, chunk)],
                        bufs[h], sems[h])

                @pl.when(p > 0)
                def _drain_out():
                    pltpu.make_async_copy(obufs[h],
                                          out_hbm.at[pl.ds(0, chunk)],
                                          osems[h]).wait()

                carry = blocks_body(bufs[h], obufs[h], cbase, carry)
                pltpu.async_copy(obufs[h], out_hbm.at[pl.ds(cbase, chunk)],
                                 osems[h])
            return carry

        carry = lax.fori_loop(0, nfull // 2, do_pair, carry)
        for h in range(2):
            pltpu.make_async_copy(obufs[h], out_hbm.at[pl.ds(0, chunk)],
                                  osems[h]).wait()
        if rem:
            base = row0 + nfull * chunk
            pltpu.sync_copy(tensor_hbm.at[pl.ds(base, rem)],
                            buf0.at[pl.ds(0, rem)])

            def tail_row(r, c2):
                return row_apply(buf0, obuf0, r, base + r, c2[0], c2[1])

            lax.fori_loop(0, rem, tail_row, carry)
            pltpu.sync_copy(obuf0.at[pl.ds(0, rem)],
                            out_hbm.at[pl.ds(base, rem)])

    return pass2


def _finalize_tc(partials, gtarg, counts, weight, bias, mean_scale, b):
    wg, d2 = partials.shape[0] * partials.shape[1], partials.shape[2]
    d = d2 // 2

    def body(part_ref, gt_ref, cnt_ref, w_ref, b_ref, ms_ref, alpha_ref, beta_ref):
        gidx = lax.broadcasted_iota(jnp.int32, (wg, b), 1)
        onehot = (gt_ref[...] == gidx).astype(jnp.float32)
        part2d = part_ref[...].reshape(wg, d2)
        s = lax.dot_general(onehot, part2d, (((0,), (0,)), ((), ())),
                            preferred_element_type=jnp.float32)
        sx = s[:, :d]
        sx2 = s[:, d:]
        invn = 1.0 / cnt_ref[...]
        mean = sx * invn
        c = mean * ms_ref[...]
        var = sx2 * invn - 2.0 * c * mean + c * c
        alpha = w_ref[...] * lax.rsqrt(var + 1e-6)
        alpha_ref[...] = alpha
        beta_ref[...] = b_ref[...] - alpha * c

    return pl.pallas_call(
        body,
        out_shape=[
            jax.ShapeDtypeStruct((b, d), jnp.float32),
            jax.ShapeDtypeStruct((b, d), jnp.float32),
        ],
    )(partials, gtarg, counts, weight, bias, mean_scale)


def kernel(graph, tensor, batch_list, weight, bias, mean_scale):
    n, d = tensor.shape
    b = batch_list.shape[0]
    w = NC * NS
    rpw = n // w
    assert n == w * rpw and rpw % 8 == 0
    chunk = 144

    ends = jnp.cumsum(batch_list.astype(jnp.int32))
    ne_idx = jnp.where(batch_list > 0, jnp.arange(b, dtype=jnp.int32), b - 1)
    sufmin = lax.cummin(ne_idx[::-1])[::-1]
    nxt = jnp.concatenate([sufmin[1:], jnp.full((1,), b - 1, jnp.int32)])
    partials = _make_pass1(n, d, b, rpw, chunk)(tensor, ends, nxt)
    g0 = jnp.searchsorted(ends, jnp.arange(w, dtype=jnp.int32) * rpw,
                          side='right').astype(jnp.int32)
    g0 = jnp.minimum(g0 & ~7, b - GPAD)
    gtarg = (g0[:, None] + jnp.arange(GPAD, dtype=jnp.int32)[None, :]).reshape(-1, 1)
    counts = batch_list.astype(jnp.float32).reshape(b, 1)
    alpha, beta = _finalize_tc(
        partials, gtarg, counts,
        weight.reshape(1, d), bias.reshape(1, d), mean_scale.reshape(1, d), b)
    return _make_pass2(n, d, b, rpw, chunk)(tensor, ends, nxt, alpha, beta)
